# baseline (speedup 1.0000x reference)
import functools

import jax
import jax.numpy as jnp
from jax import lax
from jax.experimental import pallas as pl
from jax.experimental.pallas import tpu as pltpu
from jax.experimental.pallas import tpu_sc as plsc

NUM_CORES = 2
NUM_SUBCORES = 16
NUM_WORKERS = NUM_CORES * NUM_SUBCORES

IDX_CHUNK = 128


def _gather_sc(mols, mol_table):
    B = mols.shape[0]
    _, D = mol_table.shape
    b_per_w = B // NUM_WORKERS
    n_chunks = b_per_w // IDX_CHUNK
    idx3 = mols.reshape(NUM_WORKERS, n_chunks, IDX_CHUNK)

    mesh = plsc.VectorSubcoreMesh(core_axis_name="c", subcore_axis_name="s")

    @functools.partial(
        pl.kernel,
        mesh=mesh,
        out_type=jax.ShapeDtypeStruct((B, D), jnp.float32),
        scratch_types=[
            pltpu.VMEM((n_chunks, IDX_CHUNK), jnp.int32),
            pltpu.VMEM((b_per_w, D), jnp.float32),
            pltpu.SemaphoreType.DMA,
        ],
        compiler_params=pltpu.CompilerParams(use_tc_tiling_on_sc=False),
    )
    def gather_kernel(idx_hbm, table_hbm, out_hbm, idx_v, rows_v, sem):
        wid = lax.axis_index("s") * NUM_CORES + lax.axis_index("c")
        base = wid * b_per_w
        pltpu.sync_copy(idx_hbm.at[wid], idx_v)
        copies = []
        for j in range(n_chunks):
            copies.append(
                pltpu.make_async_copy(
                    table_hbm.at[idx_v.at[j]],
                    rows_v.at[pl.ds(j * IDX_CHUNK, IDX_CHUNK)],
                    sem,
                )
            )
        for c in copies:
            c.start()
        for c in copies:
            c.wait()
        pltpu.sync_copy(rows_v, out_hbm.at[pl.ds(base, b_per_w)])

    return gather_kernel(idx3, mol_table)


def _scores_tc(mol_vecs, task_table_t):
    B, D = mol_vecs.shape
    T = task_table_t.shape[1]
    BB = 1024

    def mm_kernel(mv_ref, tt_ref, out_ref):
        out_ref[...] = jnp.dot(
            mv_ref[...], tt_ref[...], preferred_element_type=jnp.float32
        )

    return pl.pallas_call(
        mm_kernel,
        grid=(B // BB,),
        in_specs=[
            pl.BlockSpec((BB, D), lambda i: (i, 0)),
            pl.BlockSpec((D, T), lambda i: (0, 0)),
        ],
        out_specs=pl.BlockSpec((BB, T), lambda i: (i, 0)),
        out_shape=jax.ShapeDtypeStruct((B, T), jnp.float32),
    )(mol_vecs, task_table_t)


def kernel(mols, mol_table, task_table):
    mol_vecs = _gather_sc(mols.astype(jnp.int32), mol_table)
    return _scores_tc(mol_vecs, task_table.T)
